# baseline (speedup 1.0000x reference)
import jax
import jax.numpy as jnp
from jax.experimental import pallas as pl
from jax.experimental.pallas import tpu as pltpu

_BT = 2048


def _decode_kernel(order_ref, seid_ref, x_ref, w_ref, bias_ref, o_ref):
    del order_ref, seid_ref
    xb = x_ref[0].astype(jnp.bfloat16)
    wb = w_ref[0].astype(jnp.bfloat16)
    acc = jnp.dot(xb, wb, preferred_element_type=jnp.float32)
    o_ref[0] = acc + bias_ref[0]


def kernel(x, eid, W, b):
    B, T, P = x.shape
    E, _, N = W.shape
    order = jax.lax.iota(jnp.int32, B)
    seid = eid
    grid = (B, T // _BT)
    grid_spec = pltpu.PrefetchScalarGridSpec(
        num_scalar_prefetch=2,
        grid=grid,
        in_specs=[
            pl.BlockSpec((1, _BT, P), lambda bi, ti, ordr, se: (ordr[bi], ti, 0)),
            pl.BlockSpec((1, P, N), lambda bi, ti, ordr, se: (se[bi], 0, 0)),
            pl.BlockSpec((1, 1, N), lambda bi, ti, ordr, se: (se[bi], 0, 0)),
        ],
        out_specs=pl.BlockSpec((1, _BT, N), lambda bi, ti, ordr, se: (ordr[bi], ti, 0)),
    )
    return pl.pallas_call(
        _decode_kernel,
        grid_spec=grid_spec,
        out_shape=jax.ShapeDtypeStruct((B, T, N), jnp.float32),
        compiler_params=pltpu.CompilerParams(
            dimension_semantics=("parallel", "arbitrary"),
        ),
    )(order, seid, x, W, b.reshape(E, 1, N))


# speedup vs baseline: 1.0305x; 1.0305x over previous
import jax
import jax.numpy as jnp
from jax.experimental import pallas as pl
from jax.experimental.pallas import tpu as pltpu

_BT = 2048


def _decode_kernel(order_ref, seid_ref, x_ref, w_ref, bias_ref, o_ref):
    del order_ref, seid_ref
    acc = jax.lax.dot_general(
        x_ref[0], w_ref[0], (((1,), (0,)), ((), ())),
        precision=jax.lax.Precision.DEFAULT,
        preferred_element_type=jnp.float32)
    o_ref[0] = acc + bias_ref[0]


def kernel(x, eid, W, b):
    B, T, P = x.shape
    E, _, N = W.shape
    order = jnp.argsort(eid).astype(jnp.int32)
    seid = jnp.take(eid, order)
    grid = (B, T // _BT)
    grid_spec = pltpu.PrefetchScalarGridSpec(
        num_scalar_prefetch=2,
        grid=grid,
        in_specs=[
            pl.BlockSpec((1, _BT, P), lambda bi, ti, ordr, se: (ordr[bi], ti, 0)),
            pl.BlockSpec((1, P, N), lambda bi, ti, ordr, se: (se[bi], 0, 0)),
            pl.BlockSpec((1, 1, N), lambda bi, ti, ordr, se: (se[bi], 0, 0)),
        ],
        out_specs=pl.BlockSpec((1, _BT, N), lambda bi, ti, ordr, se: (ordr[bi], ti, 0)),
    )
    return pl.pallas_call(
        _decode_kernel,
        grid_spec=grid_spec,
        out_shape=jax.ShapeDtypeStruct((B, T, N), jnp.float32),
        compiler_params=pltpu.CompilerParams(
            dimension_semantics=("parallel", "arbitrary"),
        ),
    )(order, seid, x, W, b.reshape(E, 1, N))
